# baseline (speedup 1.0000x reference)
import functools

import jax
import jax.numpy as jnp
from jax import lax
from jax.experimental import pallas as pl
from jax.experimental.pallas import tpu as pltpu
from jax.experimental.pallas import tpu_sc as plsc

_B, _F, _H, _W = 16, 8, 512, 512
_NW = 32
_RPW = _B * _H // _NW
_NPAIR = _RPW // 2
_NG = _W // 16


def _zero16f():
    return jnp.zeros((16,), jnp.float32)


def _zero16i():
    return jnp.zeros((16,), jnp.int32)


def _sc_body(in_hbm, tg_hbm, pk_hbm, out_hbm,
             iv0, tv0, pv0, iv1, tv1, pv1,
             sm_ref, sa_ref, tot_ref, cm_ref, ca_ref, st_ref,
             sem_a, sem_b):
    cid = lax.axis_index("c")
    sid = lax.axis_index("s")
    wid = sid * 2 + cid
    b = wid // 2
    h0 = (wid % 2) * (_H // 2)

    for f in range(_F):
        sm_ref[f] = _zero16f()
        sa_ref[f] = _zero16f()
        tot_ref[f] = _zero16f()
        cm_ref[f] = _zero16i()
    ca_ref[0] = _zero16i()

    def issue(h, iv, tv, pv, sem):
        pltpu.make_async_copy(in_hbm.at[b, :, h, :], iv, sem).start()
        pltpu.make_async_copy(tg_hbm.at[b, :, h, :], tv, sem).start()
        pltpu.make_async_copy(pk_hbm.at[b, :, h, :], pv, sem).start()

    def drain(iv, tv, pv, sem):
        pltpu.make_async_copy(in_hbm.at[b, :, 0, :], iv, sem).wait()
        pltpu.make_async_copy(tg_hbm.at[b, :, 0, :], tv, sem).wait()
        pltpu.make_async_copy(pk_hbm.at[b, :, 0, :], pv, sem).wait()

    def compute(iv, tv, pv):
        ws = [pv[f] for f in range(_F)]
        any_w = ws[0]
        for f in range(1, _F):
            any_w = any_w | ws[f]
        tots = [_zero16f() for _ in range(_F)]
        sms = [_zero16f() for _ in range(_F)]
        sas = [_zero16f() for _ in range(_F)]
        cms = [_zero16i() for _ in range(_F)]
        cas = _zero16i()
        for g in range(_NG):
            a_bit = (any_w >> g) & 1
            af = a_bit.astype(jnp.float32)
            cas = cas + a_bit
            for f in range(_F):
                m_bit = (ws[f] >> g) & 1
                mf = m_bit.astype(jnp.float32)
                d = jnp.abs(iv[f, pl.ds(16 * g, 16)] -
                            tv[f, pl.ds(16 * g, 16)])
                tots[f] = tots[f] + d
                cms[f] = cms[f] + m_bit
                sms[f] = sms[f] + d * mf
                sas[f] = sas[f] + d * af
        for f in range(_F):
            plsc.addupdate(sm_ref.at[f], sms[f])
            plsc.addupdate(sa_ref.at[f], sas[f])
            plsc.addupdate(tot_ref.at[f], tots[f])
            plsc.addupdate(cm_ref.at[f], cms[f])
        plsc.addupdate(ca_ref.at[0], cas)

    issue(h0, iv0, tv0, pv0, sem_a)

    def body(p, carry):
        c0 = 2 * p

        @pl.when(c0 + 1 < _RPW)
        def _():
            issue(h0 + c0 + 1, iv1, tv1, pv1, sem_b)

        drain(iv0, tv0, pv0, sem_a)
        compute(iv0, tv0, pv0)

        @pl.when(c0 + 2 < _RPW)
        def _():
            issue(h0 + c0 + 2, iv0, tv0, pv0, sem_a)

        drain(iv1, tv1, pv1, sem_b)
        compute(iv1, tv1, pv1)
        return carry

    lax.fori_loop(0, _NPAIR, body, 0)

    for f in range(_F):
        st_ref[0, f] = sm_ref[f]
        st_ref[1, f] = cm_ref[f].astype(jnp.float32)
        st_ref[2, f] = tot_ref[f]
        st_ref[3, f] = sa_ref[f]
        st_ref[4, f] = _zero16f()
    st_ref[4, 0] = ca_ref[0].astype(jnp.float32)
    pltpu.sync_copy(st_ref, out_hbm.at[wid])


_MESH = plsc.VectorSubcoreMesh(core_axis_name="c", subcore_axis_name="s",
                               num_cores=2, num_subcores=16)

_sc_call = functools.partial(
    pl.kernel,
    out_type=jax.ShapeDtypeStruct((_NW, 5, _F, 16), jnp.float32),
    mesh=_MESH,
    scratch_types=[
        pltpu.VMEM((_F, _W), jnp.float32),
        pltpu.VMEM((_F, _W), jnp.float32),
        pltpu.VMEM((_F, 16), jnp.int32),
        pltpu.VMEM((_F, _W), jnp.float32),
        pltpu.VMEM((_F, _W), jnp.float32),
        pltpu.VMEM((_F, 16), jnp.int32),
        pltpu.VMEM((_F, 16), jnp.float32),
        pltpu.VMEM((_F, 16), jnp.float32),
        pltpu.VMEM((_F, 16), jnp.float32),
        pltpu.VMEM((_F, 16), jnp.int32),
        pltpu.VMEM((1, 16), jnp.int32),
        pltpu.VMEM((5, _F, 16), jnp.float32),
        pltpu.SemaphoreType.DMA,
        pltpu.SemaphoreType.DMA,
    ],
)(_sc_body)


@jax.jit
def kernel(input, target, masks, hull):
    del hull
    mu = masks.astype(jnp.uint32).reshape(_B, _F, _H, _NG, 16)
    weights = (jnp.uint32(1) << jnp.arange(_NG, dtype=jnp.uint32))
    packed = jnp.sum(mu * weights.reshape(1, 1, 1, _NG, 1), axis=3,
                     dtype=jnp.uint32)
    pk = lax.bitcast_convert_type(packed, jnp.int32)

    parts = _sc_call(input, target, pk)

    s_mask = jnp.sum(parts[:, 0], axis=(0, 2))
    c_mask = jnp.sum(parts[:, 1], axis=(0, 2))
    s_tot = jnp.sum(parts[:, 2], axis=(0, 2))
    s_all = jnp.sum(parts[:, 3], axis=(0, 2))
    c_all = jnp.sum(parts[:, 4])
    s_not = s_tot - s_mask
    c_not = float(_B * _H * _W) - c_mask

    def mmean(s, c):
        return jnp.where(c > 0, s / jnp.maximum(c, 1.0), jnp.zeros_like(s))

    lf = jnp.mean(mmean(s_mask, c_mask))
    lb = jnp.mean(mmean(s_not, c_not))
    la = jnp.mean(mmean(s_all, jnp.full_like(s_all, c_all)))
    return (lf + la + lb) / 3.0


# speedup vs baseline: 3.3496x; 3.3496x over previous
import jax
import jax.numpy as jnp
from jax.experimental import pallas as pl

_B, _F, _H, _W = 16, 8, 512, 512
_HB = 256


def _body(in_ref, tg_ref, mb_ref, out_ref):
    i = pl.program_id(0)

    @pl.when(i == 0)
    def _init():
        out_ref[...] = jnp.zeros_like(out_ref)

    mi = mb_ref[0].astype(jnp.int32)
    shifts = jax.lax.broadcasted_iota(jnp.int32, (_F, 1, 1), 0)
    m = ((mi[None] >> shifts) & 1).astype(jnp.float32)
    anym = (mi[None] != 0).astype(jnp.float32)

    a = jnp.abs(in_ref[0] - tg_ref[0])
    am = a * m

    out_ref[0] += jnp.sum(am, axis=1)
    out_ref[1] += jnp.sum(m, axis=1)
    out_ref[2] += jnp.sum(a - am, axis=1)
    out_ref[3] += jnp.sum(a * anym, axis=1)
    out_ref[4, :1] += jnp.sum(anym, axis=1)


@jax.jit
def kernel(input, target, masks, hull):
    del hull
    weights = (1 << jnp.arange(_F, dtype=jnp.int32)).reshape(1, _F, 1, 1)
    mbits = jnp.sum(masks * weights, axis=1).astype(jnp.uint8)

    grid = (_B * (_H // _HB),)
    nh = _H // _HB

    def im4(i):
        return (i // nh, 0, i % nh, 0)

    def im3(i):
        return (i // nh, i % nh, 0)

    acc = pl.pallas_call(
        _body,
        grid=grid,
        in_specs=[
            pl.BlockSpec((1, _F, _HB, _W), im4),
            pl.BlockSpec((1, _F, _HB, _W), im4),
            pl.BlockSpec((1, _HB, _W), im3),
        ],
        out_specs=pl.BlockSpec((5, _F, _W), lambda i: (0, 0, 0)),
        out_shape=jax.ShapeDtypeStruct((5, _F, _W), jnp.float32),
    )(input, target, mbits)

    sums = jnp.sum(acc, axis=-1)
    s_mask, c_mask, s_not, s_all = sums[0], sums[1], sums[2], sums[3]
    c_all = jnp.sum(acc[4, 0])
    c_not = float(_B * _H * _W) - c_mask

    def mmean(s, c):
        return jnp.where(c > 0, s / jnp.maximum(c, 1.0), jnp.zeros_like(s))

    lf = jnp.mean(mmean(s_mask, c_mask))
    lb = jnp.mean(mmean(s_not, c_not))
    la = jnp.mean(mmean(s_all, jnp.full_like(s_all, c_all)))
    return (lf + la + lb) / 3.0
